# Initial kernel scaffold; baseline (speedup 1.0000x reference)
import jax
import jax.numpy as jnp
from jax.experimental import pallas as pl


def kernel(node_feature, batch_node_ids, batch_macro_node_ids):
    raise NotImplementedError("write your pallas kernel here")


# speedup vs baseline: 5.0890x; 5.0890x over previous
import jax
import jax.numpy as jnp
from jax import lax
from jax.experimental import pallas as pl
from jax.experimental.pallas import tpu as pltpu
from jax.experimental.pallas import tpu_sc as plsc

NSEG = 20000
ROWS = 320000
D = 128
DQ = 32
NTILES = 16
WIN = 128
NWIN = 157
PER_TILE = NWIN * WIN
PAD_ROWS = NTILES * PER_TILE - ROWS
ACC_ROWS = 20480
ZROWS = ACC_ROWS // NTILES
NCHUNK = 160
NUM_NCHUNKS = NSEG // NCHUNK


def _body(nf0_hbm, nf1_hbm, nf2_hbm, nf3_hbm, ids_hbm, segs_hbm, out_hbm,
          idx_v, seg_v, gbuf, ones_v, zsum_v, zcnt_v, acc_sh, cnt_sh):
    c = lax.axis_index("c")
    s = lax.axis_index("s")

    @pl.loop(0, WIN)
    def _(i):
        ones_v[i, :] = jnp.full((16,), 1.0, dtype=jnp.float32)

    @pl.loop(0, NCHUNK)
    def _(r):
        zcnt_v[r, :] = jnp.zeros((16,), dtype=jnp.float32)

    def zero_acc(with_counts):
        @pl.loop(0, NCHUNK)
        def _(r):
            for k in range(DQ // 16):
                zsum_v[r, pl.ds(k * 16, 16)] = jnp.zeros((16,),
                                                         dtype=jnp.float32)

        zbase = s * ZROWS
        @pl.loop(0, ZROWS, step=NCHUNK)
        def _(r):
            pltpu.sync_copy(zsum_v, acc_sh.at[pl.ds(zbase + r, NCHUNK)])
            if with_counts:
                pltpu.sync_copy(zcnt_v, cnt_sh.at[pl.ds(zbase + r, NCHUNK)])

    zero_acc(with_counts=True)

    pltpu.sync_copy(ids_hbm.at[s], idx_v)
    pltpu.sync_copy(segs_hbm.at[s], seg_v)

    def scatter_pass(nf_hbm, with_counts):
        @pl.loop(0, NWIN)
        def _(w):
            pltpu.sync_copy(nf_hbm.at[idx_v.at[w]], gbuf)
            pltpu.sync_copy(gbuf, acc_sh.at[seg_v.at[w]], add=True)
            if with_counts:
                pltpu.sync_copy(ones_v, cnt_sh.at[seg_v.at[w]], add=True)

    def normalize(col0):
        @pl.loop(s, NUM_NCHUNKS, step=NTILES)
        def _(j):
            r0 = j * NCHUNK
            pltpu.sync_copy(acc_sh.at[pl.ds(r0, NCHUNK)], zsum_v)
            pltpu.sync_copy(cnt_sh.at[pl.ds(r0, NCHUNK)], zcnt_v)

            @pl.loop(0, NCHUNK)
            def _(r):
                inv = 1.0 / jnp.maximum(zcnt_v[r, :], 1.0)
                for k in range(DQ // 16):
                    zsum_v[r, pl.ds(k * 16, 16)] = (
                        zsum_v[r, pl.ds(k * 16, 16)] * inv)

            pltpu.sync_copy(zsum_v,
                            out_hbm.at[pl.ds(r0, NCHUNK), pl.ds(col0, DQ)])

    def when_core(i, fn, *args):
        @pl.when(c == i)
        def _():
            fn(*args)

    plsc.subcore_barrier()
    when_core(0, scatter_pass, nf0_hbm, True)
    when_core(1, scatter_pass, nf2_hbm, True)
    plsc.subcore_barrier()
    when_core(0, normalize, 0 * DQ)
    when_core(1, normalize, 2 * DQ)
    plsc.subcore_barrier()
    zero_acc(with_counts=False)
    plsc.subcore_barrier()
    when_core(0, scatter_pass, nf1_hbm, False)
    when_core(1, scatter_pass, nf3_hbm, False)
    plsc.subcore_barrier()
    when_core(0, normalize, 1 * DQ)
    when_core(1, normalize, 3 * DQ)


def kernel(node_feature, batch_node_ids, batch_macro_node_ids):
    ids = batch_node_ids.astype(jnp.int32)
    segs = batch_macro_node_ids.astype(jnp.int32)
    ids3 = jnp.pad(ids, (0, PAD_ROWS)).reshape(NTILES, NWIN, WIN)
    segs3 = jnp.pad(segs, (0, PAD_ROWS),
                    constant_values=NSEG).reshape(NTILES, NWIN, WIN)
    nf_q = [node_feature[:, q * DQ:(q + 1) * DQ] for q in range(4)]

    mesh = plsc.VectorSubcoreMesh(core_axis_name="c", subcore_axis_name="s")
    f32 = jnp.float32
    sc_kernel = pl.kernel(
        _body,
        out_type=jax.ShapeDtypeStruct((NSEG, D), f32),
        mesh=mesh,
        compiler_params=pltpu.CompilerParams(use_tc_tiling_on_sc=False),
        scratch_types=[
            pltpu.VMEM((NWIN, WIN), jnp.int32),
            pltpu.VMEM((NWIN, WIN), jnp.int32),
            pltpu.VMEM((WIN, DQ), f32),
            pltpu.VMEM((WIN, 16), f32),
            pltpu.VMEM((NCHUNK, DQ), f32),
            pltpu.VMEM((NCHUNK, 16), f32),
            pltpu.VMEM_SHARED((ACC_ROWS, DQ), f32),
            pltpu.VMEM_SHARED((ACC_ROWS, 16), f32),
        ],
    )
    return sc_kernel(*nf_q, ids3, segs3)


# speedup vs baseline: 7.8831x; 1.5491x over previous
import jax
import jax.numpy as jnp
from jax import lax
from jax.experimental import pallas as pl
from jax.experimental.pallas import tpu as pltpu
from jax.experimental.pallas import tpu_sc as plsc

NSEG = 20000
ROWS = 320000
D = 128
DQ = 32
NTILES = 16
WIN = 128
NWIN = 157
PER_TILE = NWIN * WIN
PAD_ROWS = NTILES * PER_TILE - ROWS
ACC_ROWS = 20480
ZROWS = ACC_ROWS // NTILES
NCHUNK = 160
NUM_NCHUNKS = NSEG // NCHUNK


def _body(nf0_hbm, nf1_hbm, nf2_hbm, nf3_hbm, ids_hbm, segs_hbm, out_hbm,
          idx_v, seg_v, gbuf, ones_v, zsum_v, zcnt_v, acc_sh, cnt_sh,
          gsem0, gsem1, ssem0, ssem1, csem0, csem1):
    gsems = (gsem0, gsem1)
    ssems = (ssem0, ssem1)
    csems = (csem0, csem1)
    c = lax.axis_index("c")
    s = lax.axis_index("s")

    @pl.loop(0, WIN)
    def _(i):
        ones_v[i, :] = jnp.full((16,), 1.0, dtype=jnp.float32)

    @pl.loop(0, NCHUNK)
    def _(r):
        zcnt_v[r, :] = jnp.zeros((16,), dtype=jnp.float32)

    def zero_acc(with_counts):
        @pl.loop(0, NCHUNK)
        def _(r):
            for k in range(DQ // 16):
                zsum_v[r, pl.ds(k * 16, 16)] = jnp.zeros((16,),
                                                         dtype=jnp.float32)

        zbase = s * ZROWS
        @pl.loop(0, ZROWS, step=NCHUNK)
        def _(r):
            pltpu.sync_copy(zsum_v, acc_sh.at[pl.ds(zbase + r, NCHUNK)])
            if with_counts:
                pltpu.sync_copy(zcnt_v, cnt_sh.at[pl.ds(zbase + r, NCHUNK)])

    zero_acc(with_counts=True)

    pltpu.sync_copy(ids_hbm.at[s], idx_v)
    pltpu.sync_copy(segs_hbm.at[s], seg_v)

    def scatter_pass(nf_hbm, with_counts):
        def g_start(w, b):
            pltpu.async_copy(nf_hbm.at[idx_v.at[w]], gbuf.at[b], gsems[b])

        def g_wait(w, b):
            pltpu.make_async_copy(nf_hbm.at[idx_v.at[w]], gbuf.at[b],
                                  gsems[b]).wait()

        def do_window(w, b):
            g_wait(w, b)
            sd = pltpu.async_copy(gbuf.at[b], acc_sh.at[seg_v.at[w]],
                                  ssems[b], add=True)
            if with_counts:
                cd = pltpu.async_copy(ones_v, cnt_sh.at[seg_v.at[w]],
                                      csems[b], add=True)
                cd.wait()
            sd.wait()

        g_start(0, 0)
        g_start(1, 1)

        @pl.loop(0, NWIN - 1, step=2)
        def _(w):
            do_window(w, 0)
            g_start(w + 2, 0)

            do_window(w + 1, 1)
            @pl.when(w + 3 < NWIN)
            def _():
                g_start(w + 3, 1)

        do_window(NWIN - 1, 0)

    def normalize(col0):
        @pl.loop(s, NUM_NCHUNKS, step=NTILES)
        def _(j):
            r0 = j * NCHUNK
            pltpu.sync_copy(acc_sh.at[pl.ds(r0, NCHUNK)], zsum_v)
            pltpu.sync_copy(cnt_sh.at[pl.ds(r0, NCHUNK)], zcnt_v)

            @pl.loop(0, NCHUNK)
            def _(r):
                inv = 1.0 / jnp.maximum(zcnt_v[r, :], 1.0)
                for k in range(DQ // 16):
                    zsum_v[r, pl.ds(k * 16, 16)] = (
                        zsum_v[r, pl.ds(k * 16, 16)] * inv)

            pltpu.sync_copy(zsum_v,
                            out_hbm.at[pl.ds(r0, NCHUNK), pl.ds(col0, DQ)])

    def when_core(i, fn, *args):
        @pl.when(c == i)
        def _():
            fn(*args)

    plsc.subcore_barrier()
    when_core(0, scatter_pass, nf0_hbm, True)
    when_core(1, scatter_pass, nf2_hbm, True)
    plsc.subcore_barrier()
    when_core(0, normalize, 0 * DQ)
    when_core(1, normalize, 2 * DQ)
    plsc.subcore_barrier()
    zero_acc(with_counts=False)
    plsc.subcore_barrier()
    when_core(0, scatter_pass, nf1_hbm, False)
    when_core(1, scatter_pass, nf3_hbm, False)
    plsc.subcore_barrier()
    when_core(0, normalize, 1 * DQ)
    when_core(1, normalize, 3 * DQ)


def kernel(node_feature, batch_node_ids, batch_macro_node_ids):
    ids = batch_node_ids.astype(jnp.int32)
    segs = batch_macro_node_ids.astype(jnp.int32)
    ids3 = jnp.pad(ids, (0, PAD_ROWS)).reshape(NTILES, NWIN, WIN)
    segs3 = jnp.pad(segs, (0, PAD_ROWS),
                    constant_values=NSEG).reshape(NTILES, NWIN, WIN)
    nf_q = [node_feature[:, q * DQ:(q + 1) * DQ] for q in range(4)]

    mesh = plsc.VectorSubcoreMesh(core_axis_name="c", subcore_axis_name="s")
    f32 = jnp.float32
    sc_kernel = pl.kernel(
        _body,
        out_type=jax.ShapeDtypeStruct((NSEG, D), f32),
        mesh=mesh,
        compiler_params=pltpu.CompilerParams(use_tc_tiling_on_sc=False),
        scratch_types=[
            pltpu.VMEM((NWIN, WIN), jnp.int32),
            pltpu.VMEM((NWIN, WIN), jnp.int32),
            pltpu.VMEM((2, WIN, DQ), f32),
            pltpu.VMEM((WIN, 16), f32),
            pltpu.VMEM((NCHUNK, DQ), f32),
            pltpu.VMEM((NCHUNK, 16), f32),
            pltpu.VMEM_SHARED((ACC_ROWS, DQ), f32),
            pltpu.VMEM_SHARED((ACC_ROWS, 16), f32),
            pltpu.SemaphoreType.DMA,
            pltpu.SemaphoreType.DMA,
            pltpu.SemaphoreType.DMA,
            pltpu.SemaphoreType.DMA,
            pltpu.SemaphoreType.DMA,
            pltpu.SemaphoreType.DMA,
        ],
    )
    return sc_kernel(*nf_q, ids3, segs3)
